# baseline (speedup 1.0000x reference)
import functools

import jax
import jax.numpy as jnp
from jax import lax
from jax.experimental import pallas as pl
from jax.experimental.pallas import tpu as pltpu
from jax.experimental.pallas import tpu_sc as plsc

_NC = 2
_NS = 16
_NW = _NC * _NS

_B = 1024
_P = 1025
_D = 64
_ROWS = 32
_QCH = 4
_CHD = _D // _QCH
_CHW = _CHD * _B


def _compute_chunk(buf, inp_v, tab_v, tab_base, d0):
    tvec = tab_v[pl.ds(tab_base + d0, _CHD)]
    for dd in range(_CHD):
        splat = jnp.full((16,), tvec[dd], jnp.float32)

        def bb_body(bb, _, dd=dd, splat=splat):
            for u in range(4):
                off = (bb * 4 + u) * 16
                buf[dd, pl.ds(off, 16)] = inp_v[d0 + dd, pl.ds(off, 16)] + splat
            return 0

        lax.fori_loop(0, _B // (16 * 4), bb_body, 0)


def _sc_body(inp_hbm, tab_hbm, out_hbm, inp_v, tab_v, tabt_v, buf0, buf1,
             sem0, sem1):
    w = lax.axis_index("s") * _NC + lax.axis_index("c")
    p0 = w * _ROWS
    pltpu.sync_copy(inp_hbm, inp_v)
    pltpu.sync_copy(tab_hbm.at[pl.ds(p0 * _D, (_ROWS + 1) * _D)], tab_v)
    pltpu.sync_copy(tab_hbm.at[pl.ds(_P * _D - _D, _D)], tabt_v)

    def fire(t, buf, sem):
        dst = out_hbm.at[p0 + t // _QCH, pl.ds((t % _QCH) * _CHD, _CHD), :]
        pltpu.async_copy(buf, dst, sem)

    def wait(t, buf, sem):
        dst = out_hbm.at[p0 + t // _QCH, pl.ds((t % _QCH) * _CHD, _CHD), :]
        pltpu.make_async_copy(buf, dst, sem).wait()

    n_t = _ROWS * _QCH

    def pair(g, _):
        t0 = 2 * g
        t1 = 2 * g + 1

        @pl.when(g > 0)
        def _():
            wait(t0 - 2, buf0, sem0)

        _compute_chunk(buf0, inp_v, tab_v, (t0 // _QCH) * _D,
                       (t0 % _QCH) * _CHD)
        fire(t0, buf0, sem0)

        @pl.when(g > 0)
        def _():
            wait(t1 - 2, buf1, sem1)

        _compute_chunk(buf1, inp_v, tab_v, (t1 // _QCH) * _D,
                       (t1 % _QCH) * _CHD)
        fire(t1, buf1, sem1)
        return 0

    lax.fori_loop(0, n_t // 2, pair, 0)
    wait(n_t - 2, buf0, sem0)
    wait(n_t - 1, buf1, sem1)

    @pl.when(w == 0)
    def _():
        def tail(q, _):
            _compute_chunk(buf0, inp_v, tabt_v, 0, q * _CHD)
            dst = out_hbm.at[_P - 1, pl.ds(q * _CHD, _CHD), :]
            pltpu.async_copy(buf0, dst, sem0).wait()
            return 0

        lax.fori_loop(0, _QCH, tail, 0)


def sc_kernel(inputs, table):
    B = inputs.shape[0]
    P, D = table.shape
    inp_t = inputs.reshape(B, D).T
    tab_flat = table.reshape(P * D)
    mesh = plsc.VectorSubcoreMesh(core_axis_name="c", subcore_axis_name="s")
    run = functools.partial(
        pl.kernel,
        mesh=mesh,
        out_type=jax.ShapeDtypeStruct((P, D, B), jnp.float32),
        scratch_types=[
            pltpu.VMEM((D, B), jnp.float32),
            pltpu.VMEM(((_ROWS + 1) * D,), jnp.float32),
            pltpu.VMEM((D,), jnp.float32),
            pltpu.VMEM((_CHD, B), jnp.float32),
            pltpu.VMEM((_CHD, B), jnp.float32),
            pltpu.SemaphoreType.DMA,
            pltpu.SemaphoreType.DMA,
        ],
    )(_sc_body)
    phys = run(inp_t, tab_flat)
    return phys.transpose(2, 0, 1)


kernel = sc_kernel


# speedup vs baseline: 5.5200x; 5.5200x over previous
import functools

import jax
import jax.numpy as jnp
from jax import lax
from jax.experimental import pallas as pl
from jax.experimental.pallas import tpu as pltpu
from jax.experimental.pallas import tpu_sc as plsc

_NC = 2
_NS = 16
_NW = _NC * _NS

_B = 1024
_P = 1025
_D = 64
_ROWS = 32
_QCH = 4
_CHD = _D // _QCH
_CHW = _CHD * _B


def _compute_chunk(buf, inp_v, tab_v, tab_base, d0):
    tvec = tab_v[pl.ds(tab_base + d0, _CHD)]
    for dd in range(_CHD):
        splat = jnp.full((16,), tvec[dd], jnp.float32)

        @plsc.parallel_loop(0, _B, step=16, unroll=8)
        def _bb(off, dd=dd, splat=splat):
            buf[dd, pl.ds(off, 16)] = inp_v[d0 + dd, pl.ds(off, 16)] + splat


def _sc_body(inp_hbm, tab_hbm, out_hbm, inp_v, tab_v, tabt_v, buf0, buf1,
             sem0, sem1):
    w = lax.axis_index("s") * _NC + lax.axis_index("c")
    p0 = w * _ROWS
    pltpu.sync_copy(inp_hbm, inp_v)
    pltpu.sync_copy(tab_hbm.at[pl.ds(p0 * _D, (_ROWS + 1) * _D)], tab_v)
    pltpu.sync_copy(tab_hbm.at[pl.ds(_P * _D - _D, _D)], tabt_v)

    def fire(t, buf, sem):
        dst = out_hbm.at[p0 + t // _QCH, pl.ds((t % _QCH) * _CHD, _CHD), :]
        pltpu.async_copy(buf, dst, sem)

    def wait(t, buf, sem):
        dst = out_hbm.at[p0 + t // _QCH, pl.ds((t % _QCH) * _CHD, _CHD), :]
        pltpu.make_async_copy(buf, dst, sem).wait()

    n_t = _ROWS * _QCH

    def pair(g, _):
        t0 = 2 * g
        t1 = 2 * g + 1

        @pl.when(g > 0)
        def _():
            wait(t0 - 2, buf0, sem0)

        _compute_chunk(buf0, inp_v, tab_v, (t0 // _QCH) * _D,
                       (t0 % _QCH) * _CHD)
        fire(t0, buf0, sem0)

        @pl.when(g > 0)
        def _():
            wait(t1 - 2, buf1, sem1)

        _compute_chunk(buf1, inp_v, tab_v, (t1 // _QCH) * _D,
                       (t1 % _QCH) * _CHD)
        fire(t1, buf1, sem1)
        return 0

    lax.fori_loop(0, n_t // 2, pair, 0)
    wait(n_t - 2, buf0, sem0)
    wait(n_t - 1, buf1, sem1)

    @pl.when(w == 0)
    def _():
        def tail(q, _):
            _compute_chunk(buf0, inp_v, tabt_v, 0, q * _CHD)
            dst = out_hbm.at[_P - 1, pl.ds(q * _CHD, _CHD), :]
            pltpu.async_copy(buf0, dst, sem0).wait()
            return 0

        lax.fori_loop(0, _QCH, tail, 0)


def sc_kernel(inputs, table):
    B = inputs.shape[0]
    P, D = table.shape
    inp_t = inputs.reshape(B, D).T
    tab_flat = table.reshape(P * D)
    mesh = plsc.VectorSubcoreMesh(core_axis_name="c", subcore_axis_name="s")
    run = functools.partial(
        pl.kernel,
        mesh=mesh,
        out_type=jax.ShapeDtypeStruct((P, D, B), jnp.float32),
        scratch_types=[
            pltpu.VMEM((D, B), jnp.float32),
            pltpu.VMEM(((_ROWS + 1) * D,), jnp.float32),
            pltpu.VMEM((D,), jnp.float32),
            pltpu.VMEM((_CHD, B), jnp.float32),
            pltpu.VMEM((_CHD, B), jnp.float32),
            pltpu.SemaphoreType.DMA,
            pltpu.SemaphoreType.DMA,
        ],
    )(_sc_body)
    phys = run(inp_t, tab_flat)
    return phys.transpose(2, 0, 1)


kernel = sc_kernel
